# Initial kernel scaffold; baseline (speedup 1.0000x reference)
import jax
import jax.numpy as jnp
from jax.experimental import pallas as pl


def kernel(node_presentation, edges, W1, b1, W2, b2):
    raise NotImplementedError("write your pallas kernel here")


# speedup vs baseline: 6.3016x; 6.3016x over previous
import functools

import jax
import jax.numpy as jnp
from jax import lax
from jax.experimental import pallas as pl
from jax.experimental.pallas import tpu as pltpu
from jax.experimental.pallas import tpu_sc as plsc

_NC = 2
_NS = 16
_NW = _NC * _NS
_K = 80


def _mm_body(x_ref, w_ref, b_ref, o_ref):
    o_ref[...] = (
        jnp.dot(x_ref[...], w_ref[...], preferred_element_type=jnp.float32)
        + b_ref[...]
    )


def _matmul_bias(x, w, b, block_rows=512):
    n, d_in = x.shape
    d_out = w.shape[1]
    return pl.pallas_call(
        _mm_body,
        grid=(pl.cdiv(n, block_rows),),
        in_specs=[
            pl.BlockSpec((block_rows, d_in), lambda i: (i, 0)),
            pl.BlockSpec((d_in, d_out), lambda i: (0, 0)),
            pl.BlockSpec((1, d_out), lambda i: (0, 0)),
        ],
        out_specs=pl.BlockSpec((block_rows, d_out), lambda i: (i, 0)),
        out_shape=jax.ShapeDtypeStruct((n, d_out), jnp.float32),
    )(x, w, b.reshape(1, d_out))


def _norm_mm_body(a0_ref, a1_ref, d0_ref, d1_ref, w_ref, b_ref, o_ref):
    deg = jnp.maximum(d0_ref[...] + d1_ref[...], 1.0)
    h = jnp.maximum((a0_ref[...] + a1_ref[...]) / deg, 0.0)
    o_ref[...] = (
        jnp.dot(h, w_ref[...], preferred_element_type=jnp.float32) + b_ref[...]
    )


def _norm_relu_matmul(a0, a1, d0, d1, w, b, block_rows=512):
    n, d_in = a0.shape
    d_out = w.shape[1]
    return pl.pallas_call(
        _norm_mm_body,
        grid=(pl.cdiv(n, block_rows),),
        in_specs=[
            pl.BlockSpec((block_rows, d_in), lambda i: (i, 0)),
            pl.BlockSpec((block_rows, d_in), lambda i: (i, 0)),
            pl.BlockSpec((block_rows, 1), lambda i: (i, 0)),
            pl.BlockSpec((block_rows, 1), lambda i: (i, 0)),
            pl.BlockSpec((d_in, d_out), lambda i: (0, 0)),
            pl.BlockSpec((1, d_out), lambda i: (0, 0)),
        ],
        out_specs=pl.BlockSpec((block_rows, d_out), lambda i: (i, 0)),
        out_shape=jax.ShapeDtypeStruct((n, d_out), jnp.float32),
    )(a0, a1, d0, d1, w, b.reshape(1, d_out))


def _norm_body(a0_ref, a1_ref, d0_ref, d1_ref, o_ref):
    deg = jnp.maximum(d0_ref[...] + d1_ref[...], 1.0)
    o_ref[...] = (a0_ref[...] + a1_ref[...]) / deg


def _norm(a0, a1, d0, d1, block_rows=512):
    n, d = a0.shape
    return pl.pallas_call(
        _norm_body,
        grid=(pl.cdiv(n, block_rows),),
        in_specs=[
            pl.BlockSpec((block_rows, d), lambda i: (i, 0)),
            pl.BlockSpec((block_rows, d), lambda i: (i, 0)),
            pl.BlockSpec((block_rows, 1), lambda i: (i, 0)),
            pl.BlockSpec((block_rows, 1), lambda i: (i, 0)),
        ],
        out_specs=pl.BlockSpec((block_rows, d), lambda i: (i, 0)),
        out_shape=jax.ShapeDtypeStruct((n, d), jnp.float32),
    )(a0, a1, d0, d1)


def _sc_aggregate(h2, src3, dst3, z_nd):
    n, d = h2.shape
    n_chunks = src3.shape[1]
    rows_per_tile = n // _NS
    mesh = plsc.VectorSubcoreMesh(core_axis_name="c", subcore_axis_name="s")

    out_type = [jax.ShapeDtypeStruct((_NC, _NS, rows_per_tile, d), jnp.float32)]
    scratch = [
        pltpu.VMEM_SHARED((n, d), jnp.float32),
        pltpu.VMEM((n_chunks, _K), jnp.int32),
        pltpu.VMEM((n_chunks, _K), jnp.int32),
        pltpu.VMEM((_K, d), jnp.float32),
        pltpu.SemaphoreType.DMA,
    ]

    def body(h2_ref, src_ref, dst_ref, z_nd_ref,
             acc_out, acc_sh, src_v, dst_v, rows_v, sem):
        c = lax.axis_index("c")
        s = lax.axis_index("s")
        wid = c * _NS + s
        row0 = s * rows_per_tile
        pltpu.sync_copy(z_nd_ref, acc_sh.at[pl.ds(row0, rows_per_tile)])
        pltpu.sync_copy(src_ref.at[wid], src_v)
        pltpu.sync_copy(dst_ref.at[wid], dst_v)
        plsc.subcore_barrier()

        def chunk(j, carry):
            pltpu.async_copy(h2_ref.at[src_v.at[j]], rows_v, sem).wait()
            pltpu.sync_copy(rows_v, acc_sh.at[dst_v.at[j]], add=True)
            return carry

        lax.fori_loop(0, n_chunks, chunk, 0)
        plsc.subcore_barrier()
        pltpu.sync_copy(acc_sh.at[pl.ds(row0, rows_per_tile)],
                        acc_out.at[c, s])

    f = pl.kernel(body, out_type=out_type, mesh=mesh, scratch_types=scratch)
    (acc,) = f(h2, src3, dst3, z_nd)
    return acc.reshape(_NC, n, d)


def _sc_degree(dst3, n, z_nd, ones_kd):
    n_chunks = dst3.shape[1]
    d = z_nd.shape[1]
    rows_per_tile = n // _NS
    mesh = plsc.VectorSubcoreMesh(core_axis_name="c", subcore_axis_name="s")

    out_type = [jax.ShapeDtypeStruct((_NC, _NS, rows_per_tile, d), jnp.float32)]
    scratch = [
        pltpu.VMEM_SHARED((n, d), jnp.float32),
        pltpu.VMEM((n_chunks, _K), jnp.int32),
        pltpu.VMEM((_K, d), jnp.float32),
    ]

    def body(dst_ref, z_nd_ref, ones_ref, deg_out, deg_sh, dst_v, ones_v):
        c = lax.axis_index("c")
        s = lax.axis_index("s")
        wid = c * _NS + s
        row0 = s * rows_per_tile
        pltpu.sync_copy(z_nd_ref, deg_sh.at[pl.ds(row0, rows_per_tile)])
        pltpu.sync_copy(ones_ref, ones_v)
        pltpu.sync_copy(dst_ref.at[wid], dst_v)
        plsc.subcore_barrier()

        def chunk(j, carry):
            pltpu.sync_copy(ones_v, deg_sh.at[dst_v.at[j]], add=True)
            return carry

        lax.fori_loop(0, n_chunks, chunk, 0)
        plsc.subcore_barrier()
        pltpu.sync_copy(deg_sh.at[pl.ds(row0, rows_per_tile)],
                        deg_out.at[c, s])

    f = pl.kernel(body, out_type=out_type, mesh=mesh, scratch_types=scratch)
    (deg,) = f(dst3, z_nd, ones_kd)
    return deg.reshape(_NC, n, d)


def kernel(node_presentation, edges, W1, b1, W2, b2):
    x = node_presentation
    n, _ = x.shape
    e = edges.shape[0]
    assert e % (_NW * _K) == 0 and n % _NS == 0
    n_chunks = e // (_NW * _K)

    src3 = edges[:, 0].astype(jnp.int32).reshape(_NW, n_chunks, _K)
    dst3 = edges[:, 1].astype(jnp.int32).reshape(_NW, n_chunks, _K)
    d_hid = W1.shape[1]
    z_nd = jnp.zeros((n // _NS, d_hid), jnp.float32)
    ones_kd = jnp.ones((_K, d_hid), jnp.float32)

    h2 = _matmul_bias(x, W1, b1)
    acc1 = _sc_aggregate(h2, src3, dst3, z_nd)
    deg = _sc_degree(dst3, n, z_nd, ones_kd)
    d0 = deg[0, :, :1]
    d1 = deg[1, :, :1]
    h2b = _norm_relu_matmul(acc1[0], acc1[1], d0, d1, W2, b2)
    acc2 = _sc_aggregate(h2b, src3, dst3, z_nd)
    return _norm(acc2[0], acc2[1], d0, d1)
